# baseline (speedup 1.0000x reference)
import functools

import jax
import jax.numpy as jnp
from jax import lax
from jax.experimental import pallas as pl
from jax.experimental.pallas import tpu as pltpu
from jax.experimental.pallas import tpu_sc as plsc

_B = 64
_CHUNK = 64
_NW = 32
_MEGA = 8


def _sc_degree(dst2d, zeros128, ones128, npad, h):
  chunks = dst2d.shape[0]
  cpw = chunks // _NW
  rpt = npad // 16

  mesh = plsc.VectorSubcoreMesh(core_axis_name="c", subcore_axis_name="s")

  @functools.partial(
      pl.kernel,
      out_type=jax.ShapeDtypeStruct((2, npad, h), jnp.float32),
      mesh=mesh,
      name="sc_degree",
      scratch_types=[
          pltpu.VMEM((cpw, _CHUNK), jnp.int32),
          pltpu.VMEM((_CHUNK, h), jnp.float32),
          pltpu.VMEM_SHARED((npad, h), jnp.float32),
      ],
  )
  def kern(dst_hbm, zeros_hbm, ones_hbm, out_hbm, idx_v, ones_v, accum):
    c = lax.axis_index("c")
    s = lax.axis_index("s")
    w = c * 16 + s
    pltpu.sync_copy(dst_hbm.at[pl.ds(w * cpw, cpw)], idx_v)
    pltpu.sync_copy(ones_hbm, ones_v)
    pltpu.sync_copy(zeros_hbm, accum.at[pl.ds(s * rpt, rpt)])
    plsc.subcore_barrier()

    def body(j, carry):
      pltpu.sync_copy(ones_v, accum.at[idx_v.at[j]], add=True)
      return carry

    lax.fori_loop(0, cpw, body, 0)
    plsc.subcore_barrier()
    pltpu.sync_copy(accum.at[pl.ds(s * rpt, rpt)],
                    out_hbm.at[c].at[pl.ds(s * rpt, rpt)])

  return kern(dst2d, zeros128, ones128)


def _sc_scatter(g, src2d, dst2d, zeros128, npad, h):
  chunks = src2d.shape[0]
  cpw = chunks // _NW
  rpt = npad // 16

  mesh = plsc.VectorSubcoreMesh(core_axis_name="c", subcore_axis_name="s")

  @functools.partial(
      pl.kernel,
      out_type=jax.ShapeDtypeStruct((2, npad, h), jnp.float32),
      mesh=mesh,
      name="sc_scatter",
      compiler_params=pltpu.CompilerParams(use_tc_tiling_on_sc=False),
      scratch_types=[
          pltpu.VMEM((cpw, _CHUNK), jnp.int32),
          pltpu.VMEM((cpw, _CHUNK), jnp.int32),
          pltpu.VMEM((_CHUNK, h), jnp.float32),
          pltpu.VMEM((_CHUNK, h), jnp.float32),
          pltpu.VMEM_SHARED((npad, h), jnp.float32),
          pltpu.SemaphoreType.DMA,
          pltpu.SemaphoreType.DMA,
      ],
  )
  def kern(g_hbm, src_hbm, dst_hbm, zeros_hbm, out_hbm,
           src_v, dst_v, rows0, rows1, accum, sem0, sem1):
    c = lax.axis_index("c")
    s = lax.axis_index("s")
    w = c * 16 + s
    pltpu.sync_copy(src_hbm.at[pl.ds(w * cpw, cpw)], src_v)
    pltpu.sync_copy(dst_hbm.at[pl.ds(w * cpw, cpw)], dst_v)
    pltpu.sync_copy(zeros_hbm, accum.at[pl.ds(s * rpt, rpt)])
    plsc.subcore_barrier()

    def body(t, carry):
      j0 = 2 * t
      d0 = pltpu.async_copy(g_hbm.at[src_v.at[j0]], rows0, sem0)
      d1 = pltpu.async_copy(g_hbm.at[src_v.at[j0 + 1]], rows1, sem1)
      d0.wait()
      pltpu.sync_copy(rows0, accum.at[dst_v.at[j0]], add=True)
      d1.wait()
      pltpu.sync_copy(rows1, accum.at[dst_v.at[j0 + 1]], add=True)
      return carry

    lax.fori_loop(0, cpw // 2, body, 0)
    plsc.subcore_barrier()
    pltpu.sync_copy(accum.at[pl.ds(s * rpt, rpt)],
                    out_hbm.at[c].at[pl.ds(s * rpt, rpt)])

  return kern(g, src2d, dst2d, zeros128)


_BLK = 2000


def _tc_first_g(x, wn, bn, degp, wc):
  n, _ = x.shape
  hdim = wn.shape[1]

  def body(x_ref, wn_ref, bn_ref, d_ref, wc_ref, g_ref, dinv_ref):
    deg = 1.0 + (d_ref[0, :, 0] + d_ref[1, :, 0])
    dinv = lax.rsqrt(deg)[:, None]
    h0 = (jnp.dot(x_ref[...], wn_ref[...], preferred_element_type=jnp.float32)
          + bn_ref[...])
    hw = jnp.dot(h0, wc_ref[...], preferred_element_type=jnp.float32)
    g_ref[...] = hw * dinv
    dinv_ref[...] = dinv

  return pl.pallas_call(
      body,
      grid=(n // _BLK,),
      in_specs=[
          pl.BlockSpec((_BLK, x.shape[1]), lambda i: (i, 0)),
          pl.BlockSpec(wn.shape, lambda i: (0, 0)),
          pl.BlockSpec((1, hdim), lambda i: (0, 0)),
          pl.BlockSpec((2, _BLK, hdim), lambda i: (0, i, 0)),
          pl.BlockSpec(wc.shape, lambda i: (0, 0)),
      ],
      out_specs=[
          pl.BlockSpec((_BLK, hdim), lambda i: (i, 0)),
          pl.BlockSpec((_BLK, 1), lambda i: (i, 0)),
      ],
      out_shape=[
          jax.ShapeDtypeStruct((n, hdim), jnp.float32),
          jax.ShapeDtypeStruct((n, 1), jnp.float32),
      ],
  )(x, wn, bn.reshape(1, hdim), degp, wc)


def _tc_mid(sp, g_prev, dinv, b_prev, w_next):
  n, hdim = g_prev.shape

  def body(s_ref, g_ref, d_ref, b_ref, w_ref, o_ref):
    tot = s_ref[0] + s_ref[1] + g_ref[...]
    hcur = jnp.maximum(tot * d_ref[...] + b_ref[...], 0.0)
    o_ref[...] = (
        jnp.dot(hcur, w_ref[...], preferred_element_type=jnp.float32)
        * d_ref[...])

  return pl.pallas_call(
      body,
      grid=(n // _BLK,),
      in_specs=[
          pl.BlockSpec((2, _BLK, hdim), lambda i: (0, i, 0)),
          pl.BlockSpec((_BLK, hdim), lambda i: (i, 0)),
          pl.BlockSpec((_BLK, 1), lambda i: (i, 0)),
          pl.BlockSpec((1, hdim), lambda i: (0, 0)),
          pl.BlockSpec(w_next.shape, lambda i: (0, 0)),
      ],
      out_specs=pl.BlockSpec((_BLK, hdim), lambda i: (i, 0)),
      out_shape=jax.ShapeDtypeStruct((n, hdim), jnp.float32),
  )(sp, g_prev, dinv, b_prev.reshape(1, hdim), w_next)


def _tc_final(sp, g_prev, dinv, b_prev, batch, wp1, bp1, wp2, bp2,
              wv1, bv1, wv2, bv2):
  n, hdim = g_prev.shape
  nblk = n // _BLK

  def body(s_ref, g_ref, d_ref, b_ref, bat_ref, wp1_ref, bp1_ref,
           wp2_ref, bp2_ref, wv1_ref, bv1_ref, wv2_ref, bv2_ref,
           pol_ref, val_ref, seg_ref, cnt_ref):
    i = pl.program_id(0)
    tot = s_ref[0] + s_ref[1] + g_ref[...]
    hcur = jnp.maximum(tot * d_ref[...] + b_ref[...], 0.0)
    p1 = jnp.maximum(
        jnp.dot(hcur, wp1_ref[...], preferred_element_type=jnp.float32)
        + bp1_ref[...], 0.0)
    pol_ref[...] = (
        jnp.dot(p1, wp2_ref[...], preferred_element_type=jnp.float32)
        + bp2_ref[...])
    onehot = jnp.equal(
        bat_ref[...], lax.broadcasted_iota(jnp.int32, (1, _B), 1)
    ).astype(jnp.float32)
    seg = lax.dot_general(onehot, hcur, (((0,), (0,)), ((), ())),
                          preferred_element_type=jnp.float32)
    cnt = jnp.sum(onehot, axis=0)[:, None]

    @pl.when(i == 0)
    def _():
      seg_ref[...] = jnp.zeros_like(seg_ref)
      cnt_ref[...] = jnp.zeros_like(cnt_ref)

    seg_ref[...] += seg
    cnt_ref[...] += cnt

    @pl.when(i == nblk - 1)
    def _():
      pooled = seg_ref[...] / jnp.maximum(cnt_ref[...], 1.0)
      v1 = jnp.maximum(
          jnp.dot(pooled, wv1_ref[...], preferred_element_type=jnp.float32)
          + bv1_ref[...], 0.0)
      val_ref[...] = jnp.tanh(
          jnp.dot(v1, wv2_ref[...], preferred_element_type=jnp.float32)
          + bv2_ref[...])

  return pl.pallas_call(
      body,
      grid=(nblk,),
      in_specs=[
          pl.BlockSpec((2, _BLK, hdim), lambda i: (0, i, 0)),
          pl.BlockSpec((_BLK, hdim), lambda i: (i, 0)),
          pl.BlockSpec((_BLK, 1), lambda i: (i, 0)),
          pl.BlockSpec((1, hdim), lambda i: (0, 0)),
          pl.BlockSpec((_BLK, 1), lambda i: (i, 0)),
          pl.BlockSpec(wp1.shape, lambda i: (0, 0)),
          pl.BlockSpec((1, hdim), lambda i: (0, 0)),
          pl.BlockSpec(wp2.shape, lambda i: (0, 0)),
          pl.BlockSpec((1, 1), lambda i: (0, 0)),
          pl.BlockSpec(wv1.shape, lambda i: (0, 0)),
          pl.BlockSpec((1, hdim), lambda i: (0, 0)),
          pl.BlockSpec(wv2.shape, lambda i: (0, 0)),
          pl.BlockSpec((1, 1), lambda i: (0, 0)),
      ],
      out_specs=[
          pl.BlockSpec((_BLK, 1), lambda i: (i, 0)),
          pl.BlockSpec((_B, 1), lambda i: (0, 0)),
          pl.BlockSpec((_B, hdim), lambda i: (0, 0)),
          pl.BlockSpec((_B, 1), lambda i: (0, 0)),
      ],
      out_shape=[
          jax.ShapeDtypeStruct((n, 1), jnp.float32),
          jax.ShapeDtypeStruct((_B, 1), jnp.float32),
          jax.ShapeDtypeStruct((_B, hdim), jnp.float32),
          jax.ShapeDtypeStruct((_B, 1), jnp.float32),
      ],
  )(sp, g_prev, dinv, b_prev.reshape(1, hdim), batch.reshape(n, 1),
    wp1, bp1.reshape(1, hdim), wp2, bp2.reshape(1, 1),
    wv1, bv1.reshape(1, hdim), wv2, bv2.reshape(1, 1))


@jax.jit
def kernel(x, edge_index, edge_attr, batch, Wn, bn, emb,
           Wc0, bc0, Wc1, bc1, Wc2, bc2,
           Wp1, bp1, Wp2, bp2, Wv1, bv1, Wv2, bv2):
  del edge_attr, emb
  n, _ = x.shape
  hdim = Wn.shape[1]
  e = edge_index.shape[1]

  npad = ((n + 128) // 128) * 128
  rpt = npad // 16

  epw = _NW * _CHUNK * 16
  e_pad = ((e + epw - 1) // epw) * epw
  src_p = jnp.concatenate(
      [edge_index[0], jnp.zeros((e_pad - e,), jnp.int32)])
  dst_p = jnp.concatenate(
      [edge_index[1], jnp.full((e_pad - e,), n, jnp.int32)])
  src2d = src_p.reshape(e_pad // _CHUNK, _CHUNK)
  dst2d = dst_p.reshape(e_pad // _CHUNK, _CHUNK)
  zeros128 = jnp.zeros((rpt, hdim), jnp.float32)
  ones128 = jnp.ones((_CHUNK, hdim), jnp.float32)

  degp = _sc_degree(dst2d, zeros128, ones128, npad, hdim)
  g0, dinv = _tc_first_g(x, Wn, bn, degp, Wc0)

  s0 = _sc_scatter(g0, src2d, dst2d, zeros128, npad, hdim)
  g1 = _tc_mid(s0, g0, dinv, bc0, Wc1)
  s1 = _sc_scatter(g1, src2d, dst2d, zeros128, npad, hdim)
  g2 = _tc_mid(s1, g1, dinv, bc1, Wc2)
  s2 = _sc_scatter(g2, src2d, dst2d, zeros128, npad, hdim)

  policy, value, _, _ = _tc_final(s2, g2, dinv, bc2, batch,
                                  Wp1, bp1, Wp2, bp2, Wv1, bv1, Wv2, bv2)
  return (policy, value)


# speedup vs baseline: 2.5939x; 2.5939x over previous
import functools

import jax
import jax.numpy as jnp
from jax import lax
from jax.experimental import pallas as pl
from jax.experimental.pallas import tpu as pltpu
from jax.experimental.pallas import tpu_sc as plsc

_B = 64
_CHUNK = 125
_NW = 32
_MEGA = 8


def _sc_degree(dst2d, zeros128, ones128, npad, h):
  chunks = dst2d.shape[0]
  cpw = chunks // _NW
  rpt = npad // 16

  mesh = plsc.VectorSubcoreMesh(core_axis_name="c", subcore_axis_name="s")

  @functools.partial(
      pl.kernel,
      out_type=jax.ShapeDtypeStruct((2, npad, h), jnp.float32),
      mesh=mesh,
      name="sc_degree",
      scratch_types=[
          pltpu.VMEM((cpw, _CHUNK), jnp.int32),
          pltpu.VMEM((_CHUNK, h), jnp.float32),
          pltpu.VMEM_SHARED((npad, h), jnp.float32),
      ],
  )
  def kern(dst_hbm, zeros_hbm, ones_hbm, out_hbm, idx_v, ones_v, accum):
    c = lax.axis_index("c")
    s = lax.axis_index("s")
    w = c * 16 + s
    pltpu.sync_copy(dst_hbm.at[pl.ds(w * cpw, cpw)], idx_v)
    pltpu.sync_copy(ones_hbm, ones_v)
    pltpu.sync_copy(zeros_hbm, accum.at[pl.ds(s * rpt, rpt)])
    plsc.subcore_barrier()

    def body(j, carry):
      pltpu.sync_copy(ones_v, accum.at[idx_v.at[j]], add=True)
      return carry

    lax.fori_loop(0, cpw, body, 0)
    plsc.subcore_barrier()
    pltpu.sync_copy(accum.at[pl.ds(s * rpt, rpt)],
                    out_hbm.at[c].at[pl.ds(s * rpt, rpt)])

  return kern(dst2d, zeros128, ones128)


def _sc_scatter(g, src2d, dst2d, zeros128, npad, h):
  chunks = src2d.shape[0]
  cpw = chunks // _NW
  rpt = npad // 16

  mesh = plsc.VectorSubcoreMesh(core_axis_name="c", subcore_axis_name="s")

  @functools.partial(
      pl.kernel,
      out_type=jax.ShapeDtypeStruct((2, npad, h), jnp.float32),
      mesh=mesh,
      name="sc_scatter",
      scratch_types=[
          pltpu.VMEM((cpw, _CHUNK), jnp.int32),
          pltpu.VMEM((cpw, _CHUNK), jnp.int32),
          pltpu.VMEM((_CHUNK, h), jnp.float32),
          pltpu.VMEM_SHARED((npad, h), jnp.float32),
          pltpu.SemaphoreType.DMA,
      ],
  )
  def kern(g_hbm, src_hbm, dst_hbm, zeros_hbm, out_hbm,
           src_v, dst_v, rows, accum, sem):
    c = lax.axis_index("c")
    s = lax.axis_index("s")
    w = c * 16 + s
    pltpu.sync_copy(src_hbm.at[pl.ds(w * cpw, cpw)], src_v)
    pltpu.sync_copy(dst_hbm.at[pl.ds(w * cpw, cpw)], dst_v)
    pltpu.sync_copy(zeros_hbm, accum.at[pl.ds(s * rpt, rpt)])
    plsc.subcore_barrier()

    def body(j, carry):
      pltpu.async_copy(g_hbm.at[src_v.at[j]], rows, sem).wait()
      pltpu.sync_copy(rows, accum.at[dst_v.at[j]], add=True)
      return carry

    lax.fori_loop(0, cpw, body, 0)
    plsc.subcore_barrier()
    pltpu.sync_copy(accum.at[pl.ds(s * rpt, rpt)],
                    out_hbm.at[c].at[pl.ds(s * rpt, rpt)])

  return kern(g, src2d, dst2d, zeros128)


_BLK = 2000


def _tc_first_g(x, wn, bn, degp, wc):
  n, _ = x.shape
  hdim = wn.shape[1]

  def body(x_ref, wn_ref, bn_ref, d_ref, wc_ref, g_ref, dinv_ref):
    deg = 1.0 + (d_ref[0, :, 0] + d_ref[1, :, 0])
    dinv = lax.rsqrt(deg)[:, None]
    h0 = (jnp.dot(x_ref[...], wn_ref[...], preferred_element_type=jnp.float32)
          + bn_ref[...])
    hw = jnp.dot(h0, wc_ref[...], preferred_element_type=jnp.float32)
    g_ref[...] = hw * dinv
    dinv_ref[...] = dinv

  return pl.pallas_call(
      body,
      grid=(n // _BLK,),
      in_specs=[
          pl.BlockSpec((_BLK, x.shape[1]), lambda i: (i, 0)),
          pl.BlockSpec(wn.shape, lambda i: (0, 0)),
          pl.BlockSpec((1, hdim), lambda i: (0, 0)),
          pl.BlockSpec((2, _BLK, hdim), lambda i: (0, i, 0)),
          pl.BlockSpec(wc.shape, lambda i: (0, 0)),
      ],
      out_specs=[
          pl.BlockSpec((_BLK, hdim), lambda i: (i, 0)),
          pl.BlockSpec((_BLK, 1), lambda i: (i, 0)),
      ],
      out_shape=[
          jax.ShapeDtypeStruct((n, hdim), jnp.float32),
          jax.ShapeDtypeStruct((n, 1), jnp.float32),
      ],
  )(x, wn, bn.reshape(1, hdim), degp, wc)


def _tc_mid(sp, g_prev, dinv, b_prev, w_next):
  n, hdim = g_prev.shape

  def body(s_ref, g_ref, d_ref, b_ref, w_ref, o_ref):
    tot = s_ref[0] + s_ref[1] + g_ref[...]
    hcur = jnp.maximum(tot * d_ref[...] + b_ref[...], 0.0)
    o_ref[...] = (
        jnp.dot(hcur, w_ref[...], preferred_element_type=jnp.float32)
        * d_ref[...])

  return pl.pallas_call(
      body,
      grid=(n // _BLK,),
      in_specs=[
          pl.BlockSpec((2, _BLK, hdim), lambda i: (0, i, 0)),
          pl.BlockSpec((_BLK, hdim), lambda i: (i, 0)),
          pl.BlockSpec((_BLK, 1), lambda i: (i, 0)),
          pl.BlockSpec((1, hdim), lambda i: (0, 0)),
          pl.BlockSpec(w_next.shape, lambda i: (0, 0)),
      ],
      out_specs=pl.BlockSpec((_BLK, hdim), lambda i: (i, 0)),
      out_shape=jax.ShapeDtypeStruct((n, hdim), jnp.float32),
  )(sp, g_prev, dinv, b_prev.reshape(1, hdim), w_next)


def _tc_final(sp, g_prev, dinv, b_prev, batch, wp1, bp1, wp2, bp2,
              wv1, bv1, wv2, bv2):
  n, hdim = g_prev.shape
  nblk = n // _BLK

  def body(s_ref, g_ref, d_ref, b_ref, bat_ref, wp1_ref, bp1_ref,
           wp2_ref, bp2_ref, wv1_ref, bv1_ref, wv2_ref, bv2_ref,
           pol_ref, val_ref, seg_ref, cnt_ref):
    i = pl.program_id(0)
    tot = s_ref[0] + s_ref[1] + g_ref[...]
    hcur = jnp.maximum(tot * d_ref[...] + b_ref[...], 0.0)
    p1 = jnp.maximum(
        jnp.dot(hcur, wp1_ref[...], preferred_element_type=jnp.float32)
        + bp1_ref[...], 0.0)
    pol_ref[...] = (
        jnp.dot(p1, wp2_ref[...], preferred_element_type=jnp.float32)
        + bp2_ref[...])
    onehot = jnp.equal(
        bat_ref[...], lax.broadcasted_iota(jnp.int32, (1, _B), 1)
    ).astype(jnp.float32)
    seg = lax.dot_general(onehot, hcur, (((0,), (0,)), ((), ())),
                          preferred_element_type=jnp.float32)
    cnt = jnp.sum(onehot, axis=0)[:, None]

    @pl.when(i == 0)
    def _():
      seg_ref[...] = jnp.zeros_like(seg_ref)
      cnt_ref[...] = jnp.zeros_like(cnt_ref)

    seg_ref[...] += seg
    cnt_ref[...] += cnt

    @pl.when(i == nblk - 1)
    def _():
      pooled = seg_ref[...] / jnp.maximum(cnt_ref[...], 1.0)
      v1 = jnp.maximum(
          jnp.dot(pooled, wv1_ref[...], preferred_element_type=jnp.float32)
          + bv1_ref[...], 0.0)
      val_ref[...] = jnp.tanh(
          jnp.dot(v1, wv2_ref[...], preferred_element_type=jnp.float32)
          + bv2_ref[...])

  return pl.pallas_call(
      body,
      grid=(nblk,),
      in_specs=[
          pl.BlockSpec((2, _BLK, hdim), lambda i: (0, i, 0)),
          pl.BlockSpec((_BLK, hdim), lambda i: (i, 0)),
          pl.BlockSpec((_BLK, 1), lambda i: (i, 0)),
          pl.BlockSpec((1, hdim), lambda i: (0, 0)),
          pl.BlockSpec((_BLK, 1), lambda i: (i, 0)),
          pl.BlockSpec(wp1.shape, lambda i: (0, 0)),
          pl.BlockSpec((1, hdim), lambda i: (0, 0)),
          pl.BlockSpec(wp2.shape, lambda i: (0, 0)),
          pl.BlockSpec((1, 1), lambda i: (0, 0)),
          pl.BlockSpec(wv1.shape, lambda i: (0, 0)),
          pl.BlockSpec((1, hdim), lambda i: (0, 0)),
          pl.BlockSpec(wv2.shape, lambda i: (0, 0)),
          pl.BlockSpec((1, 1), lambda i: (0, 0)),
      ],
      out_specs=[
          pl.BlockSpec((_BLK, 1), lambda i: (i, 0)),
          pl.BlockSpec((_B, 1), lambda i: (0, 0)),
          pl.BlockSpec((_B, hdim), lambda i: (0, 0)),
          pl.BlockSpec((_B, 1), lambda i: (0, 0)),
      ],
      out_shape=[
          jax.ShapeDtypeStruct((n, 1), jnp.float32),
          jax.ShapeDtypeStruct((_B, 1), jnp.float32),
          jax.ShapeDtypeStruct((_B, hdim), jnp.float32),
          jax.ShapeDtypeStruct((_B, 1), jnp.float32),
      ],
  )(sp, g_prev, dinv, b_prev.reshape(1, hdim), batch.reshape(n, 1),
    wp1, bp1.reshape(1, hdim), wp2, bp2.reshape(1, 1),
    wv1, bv1.reshape(1, hdim), wv2, bv2.reshape(1, 1))


@jax.jit
def kernel(x, edge_index, edge_attr, batch, Wn, bn, emb,
           Wc0, bc0, Wc1, bc1, Wc2, bc2,
           Wp1, bp1, Wp2, bp2, Wv1, bv1, Wv2, bv2):
  del edge_attr, emb
  n, _ = x.shape
  hdim = Wn.shape[1]
  e = edge_index.shape[1]

  npad = ((n + 128) // 128) * 128
  rpt = npad // 16

  src2d = edge_index[0].reshape(e // _CHUNK, _CHUNK)
  dst2d = edge_index[1].reshape(e // _CHUNK, _CHUNK)
  zeros128 = jnp.zeros((rpt, hdim), jnp.float32)
  ones128 = jnp.ones((_CHUNK, hdim), jnp.float32)

  degp = _sc_degree(dst2d, zeros128, ones128, npad, hdim)
  g0, dinv = _tc_first_g(x, Wn, bn, degp, Wc0)

  s0 = _sc_scatter(g0, src2d, dst2d, zeros128, npad, hdim)
  g1 = _tc_mid(s0, g0, dinv, bc0, Wc1)
  s1 = _sc_scatter(g1, src2d, dst2d, zeros128, npad, hdim)
  g2 = _tc_mid(s1, g1, dinv, bc1, Wc2)
  s2 = _sc_scatter(g2, src2d, dst2d, zeros128, npad, hdim)

  policy, value, _, _ = _tc_final(s2, g2, dinv, bc2, batch,
                                  Wp1, bp1, Wp2, bp2, Wv1, bv1, Wv2, bv2)
  return (policy, value)


# speedup vs baseline: 2.6051x; 1.0043x over previous
import functools

import jax
import jax.numpy as jnp
from jax import lax
from jax.experimental import pallas as pl
from jax.experimental.pallas import tpu as pltpu
from jax.experimental.pallas import tpu_sc as plsc

_B = 64
_CHUNK = 125
_NW = 32
_MEGA = 8


def _sc_degree(dst2d, zeros128, ones128, npad, h):
  chunks = dst2d.shape[0]
  cpw = chunks // _NW
  rpt = npad // 16

  mesh = plsc.VectorSubcoreMesh(core_axis_name="c", subcore_axis_name="s")

  @functools.partial(
      pl.kernel,
      out_type=jax.ShapeDtypeStruct((2, npad, h), jnp.float32),
      mesh=mesh,
      name="sc_degree",
      scratch_types=[
          pltpu.VMEM((cpw, _CHUNK), jnp.int32),
          pltpu.VMEM((_CHUNK, h), jnp.float32),
          pltpu.VMEM_SHARED((npad, h), jnp.float32),
      ],
  )
  def kern(dst_hbm, zeros_hbm, ones_hbm, out_hbm, idx_v, ones_v, accum):
    c = lax.axis_index("c")
    s = lax.axis_index("s")
    w = c * 16 + s
    pltpu.sync_copy(dst_hbm.at[pl.ds(w * cpw, cpw)], idx_v)
    pltpu.sync_copy(ones_hbm, ones_v)
    pltpu.sync_copy(zeros_hbm, accum.at[pl.ds(s * rpt, rpt)])
    plsc.subcore_barrier()

    def body(j, carry):
      pltpu.sync_copy(ones_v, accum.at[idx_v.at[j]], add=True)
      return carry

    lax.fori_loop(0, cpw, body, 0)
    plsc.subcore_barrier()
    pltpu.sync_copy(accum.at[pl.ds(s * rpt, rpt)],
                    out_hbm.at[c].at[pl.ds(s * rpt, rpt)])

  return kern(dst2d, zeros128, ones128)


def _sc_scatter(g, src2d, dst2d, zeros128, npad, h):
  chunks = src2d.shape[0]
  cpw = chunks // _NW
  rpt = npad // 16

  mesh = plsc.VectorSubcoreMesh(core_axis_name="c", subcore_axis_name="s")

  @functools.partial(
      pl.kernel,
      out_type=jax.ShapeDtypeStruct((2, npad, h), jnp.float32),
      mesh=mesh,
      name="sc_scatter",
      scratch_types=[
          pltpu.VMEM((cpw, _CHUNK), jnp.int32),
          pltpu.VMEM((cpw, _CHUNK), jnp.int32),
          pltpu.VMEM((_CHUNK, h), jnp.float32),
          pltpu.VMEM_SHARED((npad, h), jnp.float32),
          pltpu.SemaphoreType.DMA,
      ],
  )
  def kern(g_hbm, src_hbm, dst_hbm, zeros_hbm, out_hbm,
           src_v, dst_v, rows, accum, sem):
    c = lax.axis_index("c")
    s = lax.axis_index("s")
    w = c * 16 + s
    pltpu.sync_copy(src_hbm.at[pl.ds(w * cpw, cpw)], src_v)
    pltpu.sync_copy(dst_hbm.at[pl.ds(w * cpw, cpw)], dst_v)
    pltpu.sync_copy(zeros_hbm, accum.at[pl.ds(s * rpt, rpt)])
    plsc.subcore_barrier()

    def body(j, carry):
      pltpu.async_copy(g_hbm.at[src_v.at[j]], rows, sem).wait()
      pltpu.sync_copy(rows, accum.at[dst_v.at[j]], add=True)
      return carry

    lax.fori_loop(0, cpw, body, 0)
    plsc.subcore_barrier()
    pltpu.sync_copy(accum.at[pl.ds(s * rpt, rpt)],
                    out_hbm.at[c].at[pl.ds(s * rpt, rpt)])

  return kern(g, src2d, dst2d, zeros128)


_BLK = 5000


def _tc_first_g(x, wn, bn, degp, wc):
  n, _ = x.shape
  hdim = wn.shape[1]

  def body(x_ref, wn_ref, bn_ref, d_ref, wc_ref, g_ref, dinv_ref):
    deg = 1.0 + (d_ref[0, :, 0] + d_ref[1, :, 0])
    dinv = lax.rsqrt(deg)[:, None]
    h0 = (jnp.dot(x_ref[...], wn_ref[...], preferred_element_type=jnp.float32)
          + bn_ref[...])
    hw = jnp.dot(h0, wc_ref[...], preferred_element_type=jnp.float32)
    g_ref[...] = hw * dinv
    dinv_ref[...] = dinv

  return pl.pallas_call(
      body,
      grid=(n // _BLK,),
      in_specs=[
          pl.BlockSpec((_BLK, x.shape[1]), lambda i: (i, 0)),
          pl.BlockSpec(wn.shape, lambda i: (0, 0)),
          pl.BlockSpec((1, hdim), lambda i: (0, 0)),
          pl.BlockSpec((2, _BLK, hdim), lambda i: (0, i, 0)),
          pl.BlockSpec(wc.shape, lambda i: (0, 0)),
      ],
      out_specs=[
          pl.BlockSpec((_BLK, hdim), lambda i: (i, 0)),
          pl.BlockSpec((_BLK, 1), lambda i: (i, 0)),
      ],
      out_shape=[
          jax.ShapeDtypeStruct((n, hdim), jnp.float32),
          jax.ShapeDtypeStruct((n, 1), jnp.float32),
      ],
  )(x, wn, bn.reshape(1, hdim), degp, wc)


def _tc_mid(sp, g_prev, dinv, b_prev, w_next):
  n, hdim = g_prev.shape

  def body(s_ref, g_ref, d_ref, b_ref, w_ref, o_ref):
    tot = s_ref[0] + s_ref[1] + g_ref[...]
    hcur = jnp.maximum(tot * d_ref[...] + b_ref[...], 0.0)
    o_ref[...] = (
        jnp.dot(hcur, w_ref[...], preferred_element_type=jnp.float32)
        * d_ref[...])

  return pl.pallas_call(
      body,
      grid=(n // _BLK,),
      in_specs=[
          pl.BlockSpec((2, _BLK, hdim), lambda i: (0, i, 0)),
          pl.BlockSpec((_BLK, hdim), lambda i: (i, 0)),
          pl.BlockSpec((_BLK, 1), lambda i: (i, 0)),
          pl.BlockSpec((1, hdim), lambda i: (0, 0)),
          pl.BlockSpec(w_next.shape, lambda i: (0, 0)),
      ],
      out_specs=pl.BlockSpec((_BLK, hdim), lambda i: (i, 0)),
      out_shape=jax.ShapeDtypeStruct((n, hdim), jnp.float32),
  )(sp, g_prev, dinv, b_prev.reshape(1, hdim), w_next)


def _tc_final(sp, g_prev, dinv, b_prev, batch, wp1, bp1, wp2, bp2,
              wv1, bv1, wv2, bv2):
  n, hdim = g_prev.shape
  nblk = n // _BLK

  def body(s_ref, g_ref, d_ref, b_ref, bat_ref, wp1_ref, bp1_ref,
           wp2_ref, bp2_ref, wv1_ref, bv1_ref, wv2_ref, bv2_ref,
           pol_ref, val_ref, seg_ref, cnt_ref):
    i = pl.program_id(0)
    tot = s_ref[0] + s_ref[1] + g_ref[...]
    hcur = jnp.maximum(tot * d_ref[...] + b_ref[...], 0.0)
    p1 = jnp.maximum(
        jnp.dot(hcur, wp1_ref[...], preferred_element_type=jnp.float32)
        + bp1_ref[...], 0.0)
    pol_ref[...] = (
        jnp.dot(p1, wp2_ref[...], preferred_element_type=jnp.float32)
        + bp2_ref[...])
    onehot = jnp.equal(
        bat_ref[...], lax.broadcasted_iota(jnp.int32, (1, _B), 1)
    ).astype(jnp.float32)
    seg = lax.dot_general(onehot, hcur, (((0,), (0,)), ((), ())),
                          preferred_element_type=jnp.float32)
    cnt = jnp.sum(onehot, axis=0)[:, None]

    @pl.when(i == 0)
    def _():
      seg_ref[...] = jnp.zeros_like(seg_ref)
      cnt_ref[...] = jnp.zeros_like(cnt_ref)

    seg_ref[...] += seg
    cnt_ref[...] += cnt

    @pl.when(i == nblk - 1)
    def _():
      pooled = seg_ref[...] / jnp.maximum(cnt_ref[...], 1.0)
      v1 = jnp.maximum(
          jnp.dot(pooled, wv1_ref[...], preferred_element_type=jnp.float32)
          + bv1_ref[...], 0.0)
      val_ref[...] = jnp.tanh(
          jnp.dot(v1, wv2_ref[...], preferred_element_type=jnp.float32)
          + bv2_ref[...])

  return pl.pallas_call(
      body,
      grid=(nblk,),
      in_specs=[
          pl.BlockSpec((2, _BLK, hdim), lambda i: (0, i, 0)),
          pl.BlockSpec((_BLK, hdim), lambda i: (i, 0)),
          pl.BlockSpec((_BLK, 1), lambda i: (i, 0)),
          pl.BlockSpec((1, hdim), lambda i: (0, 0)),
          pl.BlockSpec((_BLK, 1), lambda i: (i, 0)),
          pl.BlockSpec(wp1.shape, lambda i: (0, 0)),
          pl.BlockSpec((1, hdim), lambda i: (0, 0)),
          pl.BlockSpec(wp2.shape, lambda i: (0, 0)),
          pl.BlockSpec((1, 1), lambda i: (0, 0)),
          pl.BlockSpec(wv1.shape, lambda i: (0, 0)),
          pl.BlockSpec((1, hdim), lambda i: (0, 0)),
          pl.BlockSpec(wv2.shape, lambda i: (0, 0)),
          pl.BlockSpec((1, 1), lambda i: (0, 0)),
      ],
      out_specs=[
          pl.BlockSpec((_BLK, 1), lambda i: (i, 0)),
          pl.BlockSpec((_B, 1), lambda i: (0, 0)),
          pl.BlockSpec((_B, hdim), lambda i: (0, 0)),
          pl.BlockSpec((_B, 1), lambda i: (0, 0)),
      ],
      out_shape=[
          jax.ShapeDtypeStruct((n, 1), jnp.float32),
          jax.ShapeDtypeStruct((_B, 1), jnp.float32),
          jax.ShapeDtypeStruct((_B, hdim), jnp.float32),
          jax.ShapeDtypeStruct((_B, 1), jnp.float32),
      ],
  )(sp, g_prev, dinv, b_prev.reshape(1, hdim), batch.reshape(n, 1),
    wp1, bp1.reshape(1, hdim), wp2, bp2.reshape(1, 1),
    wv1, bv1.reshape(1, hdim), wv2, bv2.reshape(1, 1))


@jax.jit
def kernel(x, edge_index, edge_attr, batch, Wn, bn, emb,
           Wc0, bc0, Wc1, bc1, Wc2, bc2,
           Wp1, bp1, Wp2, bp2, Wv1, bv1, Wv2, bv2):
  del edge_attr, emb
  n, _ = x.shape
  hdim = Wn.shape[1]
  e = edge_index.shape[1]

  npad = ((n + 128) // 128) * 128
  rpt = npad // 16

  src2d = edge_index[0].reshape(e // _CHUNK, _CHUNK)
  dst2d = edge_index[1].reshape(e // _CHUNK, _CHUNK)
  zeros128 = jnp.zeros((rpt, hdim), jnp.float32)
  ones128 = jnp.ones((_CHUNK, hdim), jnp.float32)

  degp = _sc_degree(dst2d, zeros128, ones128, npad, hdim)
  g0, dinv = _tc_first_g(x, Wn, bn, degp, Wc0)

  s0 = _sc_scatter(g0, src2d, dst2d, zeros128, npad, hdim)
  g1 = _tc_mid(s0, g0, dinv, bc0, Wc1)
  s1 = _sc_scatter(g1, src2d, dst2d, zeros128, npad, hdim)
  g2 = _tc_mid(s1, g1, dinv, bc1, Wc2)
  s2 = _sc_scatter(g2, src2d, dst2d, zeros128, npad, hdim)

  policy, value, _, _ = _tc_final(s2, g2, dinv, bc2, batch,
                                  Wp1, bp1, Wp2, bp2, Wv1, bv1, Wv2, bv2)
  return (policy, value)
